# Initial kernel scaffold; baseline (speedup 1.0000x reference)
import jax
import jax.numpy as jnp
from jax.experimental import pallas as pl


def kernel(x, emb_W, emb_b, enc1_W, enc1_asrc, enc1_adst, enc1_b, ff1_W, ff1_b, bn1_g, bn1_b, enc2_W, enc2_asrc, enc2_adst, enc2_b, ff2_W, ff2_b, bn2_g, bn2_b, enc3_W, enc3_asrc, enc3_adst, enc3_b, ff3_W, ff3_b, bn3_g, bn3_b, dec_W, dec_asrc, dec_adst, dec_b, val_W1, val_b1, val_W2, val_b2, edge_index, batch):
    raise NotImplementedError("write your pallas kernel here")


# speedup vs baseline: 7.3189x; 7.3189x over previous
import functools

import jax
import jax.numpy as jnp
from jax import lax
from jax.experimental import pallas as pl
from jax.experimental.pallas import tpu as pltpu
from jax.experimental.pallas import tpu_sc as plsc

N = 10000
D = 128
NG = 16
NH = 8
SUBD = 64
NSUB = 160
NPAD = NSUB * SUBD
NW = 32
SPW = NSUB // NW
EB = 128
ETOT = 160000 + N
EPAD = ((ETOT + EB - 1) // EB) * EB
RSP = 168
BR = 400

_f32 = jnp.float32
_i32 = jnp.int32


def _make_gat_sc(H, C):
    HC = H * C
    mesh = plsc.VectorSubcoreMesh(core_axis_name="c", subcore_axis_name="s",
                                  num_cores=2, num_subcores=16)

    def body(asd_hbm, xin_hbm, srcs_hbm, dsts_hbm, rs3_hbm,
             msinit_hbm, uinit_hbm, u_hbm,
             rs_v, srcs_v, dsts_v, asd_sg, adl_v, x_g, ms_v, u_v, sem):
        w = lax.axis_index("c") * 16 + lax.axis_index("s")
        pltpu.sync_copy(rs3_hbm, rs_v)
        rsw = rs_v[pl.ds(w * 16, 16)]

        for sci in range(SPW):
            sc = w * SPW + sci
            d0 = sc * SUBD
            e_lo = rsw[sci]
            e_hi = rsw[sci + 1]
            b_lo = e_lo // EB
            b_hi = (e_hi + EB - 1) // EB
            pltpu.sync_copy(msinit_hbm, ms_v)
            pltpu.sync_copy(uinit_hbm, u_v)
            pltpu.sync_copy(asd_hbm.at[pl.ds(d0, SUBD)], adl_v)

            def p_batch(b, carry):
                pltpu.sync_copy(srcs_hbm.at[pl.ds(b * EB, EB)], srcs_v)
                pltpu.sync_copy(dsts_hbm.at[pl.ds(b * EB, EB)],
                                dsts_v.at[pl.ds(0, EB)])
                pltpu.async_copy(asd_hbm.at[srcs_v], asd_sg, sem).wait()
                pltpu.async_copy(xin_hbm.at[srcs_v], x_g, sem).wait()

                def p_edge(j, c2):
                    dl = dsts_v[pl.ds(j, 16)][0] - d0

                    @pl.when((dl >= 0) & (dl < SUBD))
                    def _():
                        lv = asd_sg[j, 0:16] + adl_v[dl, 16:32]
                        lv = jnp.maximum(lv, 0.2 * lv)
                        mold = ms_v[dl, 0:16]
                        sold = ms_v[dl, 16:32]
                        mnew = jnp.maximum(mold, lv)
                        scale = jnp.exp(mold - mnew)
                        p = jnp.exp(lv - mnew)
                        ms_v[dl, 0:16] = mnew
                        ms_v[dl, 16:32] = sold * scale + p

                        for h in range(H):
                            sh = scale[h]
                            ph = p[h]

                            @pl.when(sh < 1.0)
                            def _(h=h, sh=sh):
                                for c in range(C // 16):
                                    hco = h * C + c * 16
                                    u_v[dl, hco:hco + 16] *= sh

                            for c in range(C // 16):
                                co = c * 16
                                hco = h * C + co
                                u_v[dl, hco:hco + 16] += (
                                    ph * x_g[j, co:co + 16])
                    return c2

                return lax.fori_loop(0, EB, p_edge, carry)

            lax.fori_loop(b_lo, b_hi, p_batch, 0)

            def norm_row(dd, carry):
                rvec = 1.0 / (ms_v[dd, 16:32] + 1e-16)
                for h in range(H):
                    rh = rvec[h]
                    for c in range(C // 16):
                        hco = h * C + c * 16
                        u_v[dd, hco:hco + 16] *= rh
                return carry

            lax.fori_loop(0, SUBD, norm_row, 0)
            pltpu.sync_copy(u_v, u_hbm.at[pl.ds(d0, SUBD)])

    return pl.kernel(
        body,
        out_type=jax.ShapeDtypeStruct((NPAD, HC), _f32),
        mesh=mesh,
        scratch_types=[
            pltpu.VMEM((NW * 16,), _i32),
            pltpu.VMEM((EB,), _i32),
            pltpu.VMEM((EB + 16,), _i32),
            pltpu.VMEM((EB, D), _f32),
            pltpu.VMEM((SUBD, D), _f32),
            pltpu.VMEM((EB, D), _f32),
            pltpu.VMEM((SUBD, 32), _f32),
            pltpu.VMEM((SUBD, HC), _f32),
            pltpu.SemaphoreType.DMA,
        ],
    )


@functools.lru_cache(maxsize=None)
def _gat_sc(H, C):
    return _make_gat_sc(H, C)


def _dot(a, b):
    return jax.lax.dot_general(a, b, (((1,), (0,)), ((), ())),
                               preferred_element_type=_f32)


def _tc_emb(x, W, b, wsd):
    def body(x_ref, W_ref, b_ref, wsd_ref, xe_ref, asd_ref):
        t = _dot(x_ref[...], W_ref[...]) + b_ref[...]
        xe = jnp.where(t >= 0, t, 0.01 * t)
        xe_ref[...] = xe
        asd_ref[...] = _dot(xe, wsd_ref[...])

    return pl.pallas_call(
        body, grid=(N // BR,),
        in_specs=[pl.BlockSpec((BR, D), lambda i: (i, 0)),
                  pl.BlockSpec((D, D), lambda i: (0, 0)),
                  pl.BlockSpec((D,), lambda i: (0,)),
                  pl.BlockSpec((D, D), lambda i: (0, 0))],
        out_specs=[pl.BlockSpec((BR, D), lambda i: (i, 0)),
                   pl.BlockSpec((BR, D), lambda i: (i, 0))],
        out_shape=[jax.ShapeDtypeStruct((N, D), _f32),
                   jax.ShapeDtypeStruct((N, D), _f32)],
    )(x, W, b, wsd)


def _tc_post1(u, Ws, b, wsd):
    def body(u_ref, Ws_ref, b_ref, wsd_ref, xo_ref, asd_ref):
        g = _dot(u_ref[...], Ws_ref[...]) * (1.0 / NH) + b_ref[...]
        xo_ref[...] = g
        asd_ref[...] = _dot(g, wsd_ref[...])

    return pl.pallas_call(
        body, grid=(N // BR,),
        in_specs=[pl.BlockSpec((BR, NH * D), lambda i: (i, 0)),
                  pl.BlockSpec((NH * D, D), lambda i: (0, 0)),
                  pl.BlockSpec((D,), lambda i: (0,)),
                  pl.BlockSpec((D, D), lambda i: (0, 0))],
        out_specs=[pl.BlockSpec((BR, D), lambda i: (i, 0)),
                   pl.BlockSpec((BR, D), lambda i: (i, 0))],
        out_shape=[jax.ShapeDtypeStruct((N, D), _f32),
                   jax.ShapeDtypeStruct((N, D), _f32)],
    )(u, Ws, b, wsd)


def _tc_ffbn_a(u, x_in, Ws, b, ffW, ffb, slope):
    def body(u_ref, xin_ref, Ws_ref, b_ref, ffW_ref, ffb_ref, y_ref, st_ref):
        g = _dot(u_ref[...], Ws_ref[...]) * (1.0 / NH) + b_ref[...]
        t = g + xin_ref[...]
        z = _dot(t, ffW_ref[...]) + ffb_ref[...]
        a = jnp.where(z >= 0, z, slope * z)
        y = a + xin_ref[...]
        y_ref[...] = y

        @pl.when(pl.program_id(0) == 0)
        def _():
            st_ref[...] = jnp.zeros_like(st_ref)

        st_ref[0:1, :] += jnp.sum(y, axis=0)[None, :]
        st_ref[1:2, :] += jnp.sum(y * y, axis=0)[None, :]

    return pl.pallas_call(
        body, grid=(N // BR,),
        in_specs=[pl.BlockSpec((BR, NH * D), lambda i: (i, 0)),
                  pl.BlockSpec((BR, D), lambda i: (i, 0)),
                  pl.BlockSpec((NH * D, D), lambda i: (0, 0)),
                  pl.BlockSpec((D,), lambda i: (0,)),
                  pl.BlockSpec((D, D), lambda i: (0, 0)),
                  pl.BlockSpec((D,), lambda i: (0,))],
        out_specs=[pl.BlockSpec((BR, D), lambda i: (i, 0)),
                   pl.BlockSpec((8, D), lambda i: (0, 0))],
        out_shape=[jax.ShapeDtypeStruct((N, D), _f32),
                   jax.ShapeDtypeStruct((8, D), _f32)],
    )(u, x_in, Ws, b, ffW, ffb)


def _tc_bn_b(y, st, g, b, wsd, whp=None):
    nout = 3 if whp is not None else 2

    def body(*refs):
        if nout == 3:
            (y_ref, st_ref, g_ref, b_ref, wsd_ref, whp_ref,
             xn_ref, asd_ref, hp_ref) = refs
        else:
            y_ref, st_ref, g_ref, b_ref, wsd_ref, xn_ref, asd_ref = refs
        m = st_ref[0:1, :] * (1.0 / N)
        v = st_ref[1:2, :] * (1.0 / N) - m * m
        xn = (y_ref[...] - m) * lax.rsqrt(v + 1e-5) * g_ref[...] + b_ref[...]
        xn_ref[...] = xn
        asd_ref[...] = _dot(xn, wsd_ref[...])
        if nout == 3:
            hp_ref[...] = _dot(xn, whp_ref[...])

    in_specs = [pl.BlockSpec((BR, D), lambda i: (i, 0)),
                pl.BlockSpec((8, D), lambda i: (0, 0)),
                pl.BlockSpec((D,), lambda i: (0,)),
                pl.BlockSpec((D,), lambda i: (0,)),
                pl.BlockSpec((D, D), lambda i: (0, 0))]
    out_specs = [pl.BlockSpec((BR, D), lambda i: (i, 0)),
                 pl.BlockSpec((BR, D), lambda i: (i, 0))]
    out_shape = [jax.ShapeDtypeStruct((N, D), _f32),
                 jax.ShapeDtypeStruct((N, D), _f32)]
    args = [y, st, g, b, wsd]
    if whp is not None:
        in_specs.append(pl.BlockSpec((D, D), lambda i: (0, 0)))
        out_specs.append(pl.BlockSpec((BR, D), lambda i: (i, 0)))
        out_shape.append(jax.ShapeDtypeStruct((N, D), _f32))
        args.append(whp)
    return pl.pallas_call(
        body, grid=(N // BR,), in_specs=in_specs,
        out_specs=out_specs, out_shape=out_shape)(*args)


def _tc_pool_value(x3, batch3, W1, b1, W2p, b2):
    nb = N // BR

    def body(x_ref, b_ref, W1_ref, b1_ref, W2_ref, b2_ref,
             sums_ref, cnt_ref, val_ref):
        @pl.when(pl.program_id(0) == 0)
        def _():
            sums_ref[...] = jnp.zeros_like(sums_ref)
            cnt_ref[...] = jnp.zeros_like(cnt_ref)

        bvec = b_ref[0, 0, :]
        S = (bvec[None, :] ==
             lax.broadcasted_iota(_i32, (NG, BR), 0)).astype(_f32)
        sums_ref[...] += _dot(S, x_ref[...])
        cnt_ref[...] += jnp.broadcast_to(jnp.sum(S, axis=1)[:, None], (NG, D))

        @pl.when(pl.program_id(0) == nb - 1)
        def _():
            vin = sums_ref[...] / jnp.maximum(cnt_ref[...], 1.0)
            hh = jnp.maximum(_dot(vin, W1_ref[...]) + b1_ref[...], 0.0)
            val_ref[...] = _dot(hh, W2_ref[...]) + b2_ref[...]

    return pl.pallas_call(
        body, grid=(nb,),
        in_specs=[pl.BlockSpec((BR, D), lambda i: (i, 0)),
                  pl.BlockSpec((1, 1, BR), lambda i: (i, 0, 0)),
                  pl.BlockSpec((D, D), lambda i: (0, 0)),
                  pl.BlockSpec((D,), lambda i: (0,)),
                  pl.BlockSpec((D, D), lambda i: (0, 0)),
                  pl.BlockSpec((1,), lambda i: (0,))],
        out_specs=[pl.BlockSpec((NG, D), lambda i: (0, 0)),
                   pl.BlockSpec((NG, D), lambda i: (0, 0)),
                   pl.BlockSpec((NG, D), lambda i: (0, 0))],
        out_shape=[jax.ShapeDtypeStruct((NG, D), _f32),
                   jax.ShapeDtypeStruct((NG, D), _f32),
                   jax.ShapeDtypeStruct((NG, D), _f32)],
    )(x3, batch3, W1, b1, W2p, b2)


def _enc_weights(W, a_src, a_dst):
    Wr = W.reshape(D, NH, D)
    ws = jnp.einsum('ikj,kj->ik', Wr, a_src)
    wd = jnp.einsum('ikj,kj->ik', Wr, a_dst)
    wsd = jnp.zeros((D, D), _f32)
    wsd = wsd.at[:, 0:NH].set(ws).at[:, 16:16 + NH].set(wd)
    Wstack = Wr.transpose(1, 0, 2).reshape(NH * D, D)
    return wsd, Wstack


def _pad_rows(a):
    return jnp.pad(a, ((0, NPAD - N), (0, 0)))


def kernel(x, emb_W, emb_b, enc1_W, enc1_asrc, enc1_adst, enc1_b, ff1_W,
           ff1_b, bn1_g, bn1_b, enc2_W, enc2_asrc, enc2_adst, enc2_b, ff2_W,
           ff2_b, bn2_g, bn2_b, enc3_W, enc3_asrc, enc3_adst, enc3_b, ff3_W,
           ff3_b, bn3_g, bn3_b, dec_W, dec_asrc, dec_adst, dec_b, val_W1,
           val_b1, val_W2, val_b2, edge_index, batch):
    loop = jnp.arange(N, dtype=edge_index.dtype)
    srcf = jnp.concatenate([edge_index[0], loop])
    dstf = jnp.concatenate([edge_index[1], loop])
    dsts, srcs = lax.sort([dstf, srcf], num_keys=1)
    srcs_p = jnp.pad(srcs, (0, EPAD - ETOT)).astype(_i32)
    dsts_p = jnp.pad(dsts, (0, EPAD - ETOT),
                     constant_values=NPAD - 1).astype(_i32)
    bounds = jnp.arange(NSUB + 1, dtype=_i32) * SUBD
    rs = jnp.searchsorted(dsts_p, bounds).astype(_i32)
    rs_idx = jnp.minimum(
        jnp.arange(NW, dtype=_i32)[:, None] * SPW
        + jnp.arange(16, dtype=_i32)[None, :], NSUB)
    rs3 = rs[rs_idx].reshape(-1)
    msinit = jnp.concatenate(
        [jnp.full((SUBD, 16), -1e30, _f32), jnp.zeros((SUBD, 16), _f32)],
        axis=1)
    uinit_enc = jnp.zeros((SUBD, NH * D), _f32)
    uinit_dec = jnp.zeros((SUBD, 16), _f32)

    wsd1, Ws1 = _enc_weights(enc1_W, enc1_asrc, enc1_adst)
    wsd2, Ws2 = _enc_weights(enc2_W, enc2_asrc, enc2_adst)
    wsd3, Ws3 = _enc_weights(enc3_W, enc3_asrc, enc3_adst)
    wsd_dec = jnp.zeros((D, D), _f32)
    wsd_dec = wsd_dec.at[:, 0].set(dec_W[:, 0] * dec_asrc[0, 0])
    wsd_dec = wsd_dec.at[:, 16].set(dec_W[:, 0] * dec_adst[0, 0])
    whp = jnp.zeros((D, D), _f32).at[:, 0].set(dec_W[:, 0])
    W2p = jnp.pad(val_W2, ((0, 0), (0, D - val_W2.shape[1])))

    def gat_enc(asd, xin):
        u = _gat_sc(NH, D)(_pad_rows(asd), _pad_rows(xin),
                        srcs_p, dsts_p, rs3, msinit, uinit_enc)
        return u[:N]

    x_e, asd = _tc_emb(x, emb_W, emb_b, wsd1)
    u = gat_enc(asd, x_e)
    x_out, asd = _tc_post1(u, Ws1, enc1_b, wsd1)
    u = gat_enc(asd, x_out)
    y, st = _tc_ffbn_a(u, x_out, Ws1, enc1_b, ff1_W, ff1_b, 0.01)
    x1, asd = _tc_bn_b(y, st, bn1_g, bn1_b, wsd2)
    u = gat_enc(asd, x1)
    y, st = _tc_ffbn_a(u, x1, Ws2, enc2_b, ff2_W, ff2_b, 0.0)
    x2, asd = _tc_bn_b(y, st, bn2_g, bn2_b, wsd3)
    u = gat_enc(asd, x2)
    y, st = _tc_ffbn_a(u, x2, Ws3, enc3_b, ff3_W, ff3_b, 0.0)
    x3, asd_dec, hpad = _tc_bn_b(y, st, bn3_g, bn3_b, wsd_dec, whp)
    u_dec = _gat_sc(1, 16)(_pad_rows(asd_dec), _pad_rows(hpad),
                        srcs_p, dsts_p, rs3, msinit, uinit_dec)
    out = u_dec[:N, :1] + dec_b
    batch3 = batch.astype(_i32).reshape(N // BR, 1, BR)
    sums, cnt, valo = _tc_pool_value(x3, batch3, val_W1, val_b1, W2p, val_b2)
    value = valo[:, :1]
    return out, value


# speedup vs baseline: 8.0061x; 1.0939x over previous
import functools

import jax
import jax.numpy as jnp
from jax import lax
from jax.experimental import pallas as pl
from jax.experimental.pallas import tpu as pltpu
from jax.experimental.pallas import tpu_sc as plsc

N = 10000
D = 128
NG = 16
NH = 8
SUBD = 64
NSUB = 160
NPAD = NSUB * SUBD
NW = 32
SPW = NSUB // NW
EB = 128
ETOT = 160000 + N
EPAD = ((ETOT + EB - 1) // EB) * EB
RSP = 168
BR = 400

_f32 = jnp.float32
_i32 = jnp.int32


def _make_gat_sc(H, C):
    HC = H * C
    mesh = plsc.VectorSubcoreMesh(core_axis_name="c", subcore_axis_name="s",
                                  num_cores=2, num_subcores=16)

    def body(asd_hbm, xin_hbm, srcs_hbm, dsts_hbm, rs3_hbm,
             msinit_hbm, uinit_hbm, u_hbm,
             rs_v, srcs_v, dsts_v, asd_sg, adl_v, x_g, ms_v, u_v, sem):
        w = lax.axis_index("c") * 16 + lax.axis_index("s")
        pltpu.sync_copy(rs3_hbm, rs_v)
        rsw = rs_v[pl.ds(w * 16, 16)]

        for sci in range(SPW):
            sc = w * SPW + sci
            d0 = sc * SUBD
            e_lo = rsw[sci]
            e_hi = rsw[sci + 1]
            b_lo = e_lo // EB
            b_hi = (e_hi + EB - 1) // EB
            pltpu.sync_copy(msinit_hbm, ms_v)
            pltpu.sync_copy(uinit_hbm, u_v)
            pltpu.sync_copy(asd_hbm.at[pl.ds(d0, SUBD)], adl_v)

            def p_batch(b, carry):
                pltpu.sync_copy(srcs_hbm.at[pl.ds(b * EB, EB)], srcs_v)
                pltpu.sync_copy(dsts_hbm.at[pl.ds(b * EB, EB)],
                                dsts_v.at[pl.ds(0, EB)])
                pltpu.async_copy(asd_hbm.at[srcs_v], asd_sg, sem).wait()
                pltpu.async_copy(xin_hbm.at[srcs_v], x_g, sem).wait()

                def p_edge(j, c2):
                    dl = dsts_v[pl.ds(j, 16)][0] - d0

                    @pl.when((dl >= 0) & (dl < SUBD))
                    def _():
                        lv = asd_sg[j, 0:16] + adl_v[dl, 16:32]
                        lv = jnp.maximum(lv, 0.2 * lv)
                        mold = ms_v[dl, 0:16]
                        sold = ms_v[dl, 16:32]
                        mnew = jnp.maximum(mold, lv)
                        scale = jnp.exp(mold - mnew)
                        p = jnp.exp(lv - mnew)
                        ms_v[dl, 0:16] = mnew
                        ms_v[dl, 16:32] = sold * scale + p

                        for h in range(H):
                            sh = scale[h]
                            ph = p[h]

                            @pl.when(sh < 1.0)
                            def _(h=h, sh=sh):
                                for c in range(C // 16):
                                    hco = h * C + c * 16
                                    u_v[dl, hco:hco + 16] *= sh

                            for c in range(C // 16):
                                co = c * 16
                                hco = h * C + co
                                plsc.addupdate(
                                    u_v.at[dl, pl.ds(hco, 16)],
                                    ph * x_g[j, co:co + 16])
                    return c2

                return lax.fori_loop(0, EB, p_edge, carry)

            lax.fori_loop(b_lo, b_hi, p_batch, 0)

            def norm_row(dd, carry):
                rvec = 1.0 / (ms_v[dd, 16:32] + 1e-16)
                for h in range(H):
                    rh = rvec[h]
                    for c in range(C // 16):
                        hco = h * C + c * 16
                        u_v[dd, hco:hco + 16] *= rh
                return carry

            lax.fori_loop(0, SUBD, norm_row, 0)
            pltpu.sync_copy(u_v, u_hbm.at[pl.ds(d0, SUBD)])

    return pl.kernel(
        body,
        out_type=jax.ShapeDtypeStruct((NPAD, HC), _f32),
        mesh=mesh,
        scratch_types=[
            pltpu.VMEM((NW * 16,), _i32),
            pltpu.VMEM((EB,), _i32),
            pltpu.VMEM((EB + 16,), _i32),
            pltpu.VMEM((EB, D), _f32),
            pltpu.VMEM((SUBD, D), _f32),
            pltpu.VMEM((EB, D), _f32),
            pltpu.VMEM((SUBD, 32), _f32),
            pltpu.VMEM((SUBD, HC), _f32),
            pltpu.SemaphoreType.DMA,
        ],
    )


@functools.lru_cache(maxsize=None)
def _gat_sc(H, C):
    return _make_gat_sc(H, C)


def _dot(a, b):
    return jax.lax.dot_general(a, b, (((1,), (0,)), ((), ())),
                               preferred_element_type=_f32)


def _tc_emb(x, W, b, wsd):
    def body(x_ref, W_ref, b_ref, wsd_ref, xe_ref, asd_ref):
        t = _dot(x_ref[...], W_ref[...]) + b_ref[...]
        xe = jnp.where(t >= 0, t, 0.01 * t)
        xe_ref[...] = xe
        asd_ref[...] = _dot(xe, wsd_ref[...])

    return pl.pallas_call(
        body, grid=(N // BR,),
        in_specs=[pl.BlockSpec((BR, D), lambda i: (i, 0)),
                  pl.BlockSpec((D, D), lambda i: (0, 0)),
                  pl.BlockSpec((D,), lambda i: (0,)),
                  pl.BlockSpec((D, D), lambda i: (0, 0))],
        out_specs=[pl.BlockSpec((BR, D), lambda i: (i, 0)),
                   pl.BlockSpec((BR, D), lambda i: (i, 0))],
        out_shape=[jax.ShapeDtypeStruct((N, D), _f32),
                   jax.ShapeDtypeStruct((N, D), _f32)],
    )(x, W, b, wsd)


def _tc_post1(u, Ws, b, wsd):
    def body(u_ref, Ws_ref, b_ref, wsd_ref, xo_ref, asd_ref):
        g = _dot(u_ref[...], Ws_ref[...]) * (1.0 / NH) + b_ref[...]
        xo_ref[...] = g
        asd_ref[...] = _dot(g, wsd_ref[...])

    return pl.pallas_call(
        body, grid=(N // BR,),
        in_specs=[pl.BlockSpec((BR, NH * D), lambda i: (i, 0)),
                  pl.BlockSpec((NH * D, D), lambda i: (0, 0)),
                  pl.BlockSpec((D,), lambda i: (0,)),
                  pl.BlockSpec((D, D), lambda i: (0, 0))],
        out_specs=[pl.BlockSpec((BR, D), lambda i: (i, 0)),
                   pl.BlockSpec((BR, D), lambda i: (i, 0))],
        out_shape=[jax.ShapeDtypeStruct((N, D), _f32),
                   jax.ShapeDtypeStruct((N, D), _f32)],
    )(u, Ws, b, wsd)


def _tc_ffbn_a(u, x_in, Ws, b, ffW, ffb, slope):
    def body(u_ref, xin_ref, Ws_ref, b_ref, ffW_ref, ffb_ref, y_ref, st_ref):
        g = _dot(u_ref[...], Ws_ref[...]) * (1.0 / NH) + b_ref[...]
        t = g + xin_ref[...]
        z = _dot(t, ffW_ref[...]) + ffb_ref[...]
        a = jnp.where(z >= 0, z, slope * z)
        y = a + xin_ref[...]
        y_ref[...] = y

        @pl.when(pl.program_id(0) == 0)
        def _():
            st_ref[...] = jnp.zeros_like(st_ref)

        st_ref[0:1, :] += jnp.sum(y, axis=0)[None, :]
        st_ref[1:2, :] += jnp.sum(y * y, axis=0)[None, :]

    return pl.pallas_call(
        body, grid=(N // BR,),
        in_specs=[pl.BlockSpec((BR, NH * D), lambda i: (i, 0)),
                  pl.BlockSpec((BR, D), lambda i: (i, 0)),
                  pl.BlockSpec((NH * D, D), lambda i: (0, 0)),
                  pl.BlockSpec((D,), lambda i: (0,)),
                  pl.BlockSpec((D, D), lambda i: (0, 0)),
                  pl.BlockSpec((D,), lambda i: (0,))],
        out_specs=[pl.BlockSpec((BR, D), lambda i: (i, 0)),
                   pl.BlockSpec((8, D), lambda i: (0, 0))],
        out_shape=[jax.ShapeDtypeStruct((N, D), _f32),
                   jax.ShapeDtypeStruct((8, D), _f32)],
    )(u, x_in, Ws, b, ffW, ffb)


def _tc_bn_b(y, st, g, b, wsd, whp=None):
    nout = 3 if whp is not None else 2

    def body(*refs):
        if nout == 3:
            (y_ref, st_ref, g_ref, b_ref, wsd_ref, whp_ref,
             xn_ref, asd_ref, hp_ref) = refs
        else:
            y_ref, st_ref, g_ref, b_ref, wsd_ref, xn_ref, asd_ref = refs
        m = st_ref[0:1, :] * (1.0 / N)
        v = st_ref[1:2, :] * (1.0 / N) - m * m
        xn = (y_ref[...] - m) * lax.rsqrt(v + 1e-5) * g_ref[...] + b_ref[...]
        xn_ref[...] = xn
        asd_ref[...] = _dot(xn, wsd_ref[...])
        if nout == 3:
            hp_ref[...] = _dot(xn, whp_ref[...])

    in_specs = [pl.BlockSpec((BR, D), lambda i: (i, 0)),
                pl.BlockSpec((8, D), lambda i: (0, 0)),
                pl.BlockSpec((D,), lambda i: (0,)),
                pl.BlockSpec((D,), lambda i: (0,)),
                pl.BlockSpec((D, D), lambda i: (0, 0))]
    out_specs = [pl.BlockSpec((BR, D), lambda i: (i, 0)),
                 pl.BlockSpec((BR, D), lambda i: (i, 0))]
    out_shape = [jax.ShapeDtypeStruct((N, D), _f32),
                 jax.ShapeDtypeStruct((N, D), _f32)]
    args = [y, st, g, b, wsd]
    if whp is not None:
        in_specs.append(pl.BlockSpec((D, D), lambda i: (0, 0)))
        out_specs.append(pl.BlockSpec((BR, D), lambda i: (i, 0)))
        out_shape.append(jax.ShapeDtypeStruct((N, D), _f32))
        args.append(whp)
    return pl.pallas_call(
        body, grid=(N // BR,), in_specs=in_specs,
        out_specs=out_specs, out_shape=out_shape)(*args)


def _tc_pool_value(x3, batch3, W1, b1, W2p, b2):
    nb = N // BR

    def body(x_ref, b_ref, W1_ref, b1_ref, W2_ref, b2_ref,
             sums_ref, cnt_ref, val_ref):
        @pl.when(pl.program_id(0) == 0)
        def _():
            sums_ref[...] = jnp.zeros_like(sums_ref)
            cnt_ref[...] = jnp.zeros_like(cnt_ref)

        bvec = b_ref[0, 0, :]
        S = (bvec[None, :] ==
             lax.broadcasted_iota(_i32, (NG, BR), 0)).astype(_f32)
        sums_ref[...] += _dot(S, x_ref[...])
        cnt_ref[...] += jnp.broadcast_to(jnp.sum(S, axis=1)[:, None], (NG, D))

        @pl.when(pl.program_id(0) == nb - 1)
        def _():
            vin = sums_ref[...] / jnp.maximum(cnt_ref[...], 1.0)
            hh = jnp.maximum(_dot(vin, W1_ref[...]) + b1_ref[...], 0.0)
            val_ref[...] = _dot(hh, W2_ref[...]) + b2_ref[...]

    return pl.pallas_call(
        body, grid=(nb,),
        in_specs=[pl.BlockSpec((BR, D), lambda i: (i, 0)),
                  pl.BlockSpec((1, 1, BR), lambda i: (i, 0, 0)),
                  pl.BlockSpec((D, D), lambda i: (0, 0)),
                  pl.BlockSpec((D,), lambda i: (0,)),
                  pl.BlockSpec((D, D), lambda i: (0, 0)),
                  pl.BlockSpec((1,), lambda i: (0,))],
        out_specs=[pl.BlockSpec((NG, D), lambda i: (0, 0)),
                   pl.BlockSpec((NG, D), lambda i: (0, 0)),
                   pl.BlockSpec((NG, D), lambda i: (0, 0))],
        out_shape=[jax.ShapeDtypeStruct((NG, D), _f32),
                   jax.ShapeDtypeStruct((NG, D), _f32),
                   jax.ShapeDtypeStruct((NG, D), _f32)],
    )(x3, batch3, W1, b1, W2p, b2)


def _enc_weights(W, a_src, a_dst):
    Wr = W.reshape(D, NH, D)
    ws = jnp.einsum('ikj,kj->ik', Wr, a_src)
    wd = jnp.einsum('ikj,kj->ik', Wr, a_dst)
    wsd = jnp.zeros((D, D), _f32)
    wsd = wsd.at[:, 0:NH].set(ws).at[:, 16:16 + NH].set(wd)
    Wstack = Wr.transpose(1, 0, 2).reshape(NH * D, D)
    return wsd, Wstack


def _pad_rows(a):
    return jnp.pad(a, ((0, NPAD - N), (0, 0)))


def kernel(x, emb_W, emb_b, enc1_W, enc1_asrc, enc1_adst, enc1_b, ff1_W,
           ff1_b, bn1_g, bn1_b, enc2_W, enc2_asrc, enc2_adst, enc2_b, ff2_W,
           ff2_b, bn2_g, bn2_b, enc3_W, enc3_asrc, enc3_adst, enc3_b, ff3_W,
           ff3_b, bn3_g, bn3_b, dec_W, dec_asrc, dec_adst, dec_b, val_W1,
           val_b1, val_W2, val_b2, edge_index, batch):
    loop = jnp.arange(N, dtype=edge_index.dtype)
    srcf = jnp.concatenate([edge_index[0], loop])
    dstf = jnp.concatenate([edge_index[1], loop])
    dsts, srcs = lax.sort([dstf, srcf], num_keys=1)
    srcs_p = jnp.pad(srcs, (0, EPAD - ETOT)).astype(_i32)
    dsts_p = jnp.pad(dsts, (0, EPAD - ETOT),
                     constant_values=NPAD - 1).astype(_i32)
    bounds = jnp.arange(NSUB + 1, dtype=_i32) * SUBD
    rs = jnp.searchsorted(dsts_p, bounds).astype(_i32)
    rs_idx = jnp.minimum(
        jnp.arange(NW, dtype=_i32)[:, None] * SPW
        + jnp.arange(16, dtype=_i32)[None, :], NSUB)
    rs3 = rs[rs_idx].reshape(-1)
    msinit = jnp.concatenate(
        [jnp.full((SUBD, 16), -1e30, _f32), jnp.zeros((SUBD, 16), _f32)],
        axis=1)
    uinit_enc = jnp.zeros((SUBD, NH * D), _f32)
    uinit_dec = jnp.zeros((SUBD, 16), _f32)

    wsd1, Ws1 = _enc_weights(enc1_W, enc1_asrc, enc1_adst)
    wsd2, Ws2 = _enc_weights(enc2_W, enc2_asrc, enc2_adst)
    wsd3, Ws3 = _enc_weights(enc3_W, enc3_asrc, enc3_adst)
    wsd_dec = jnp.zeros((D, D), _f32)
    wsd_dec = wsd_dec.at[:, 0].set(dec_W[:, 0] * dec_asrc[0, 0])
    wsd_dec = wsd_dec.at[:, 16].set(dec_W[:, 0] * dec_adst[0, 0])
    whp = jnp.zeros((D, D), _f32).at[:, 0].set(dec_W[:, 0])
    W2p = jnp.pad(val_W2, ((0, 0), (0, D - val_W2.shape[1])))

    def gat_enc(asd, xin):
        u = _gat_sc(NH, D)(_pad_rows(asd), _pad_rows(xin),
                        srcs_p, dsts_p, rs3, msinit, uinit_enc)
        return u[:N]

    x_e, asd = _tc_emb(x, emb_W, emb_b, wsd1)
    u = gat_enc(asd, x_e)
    x_out, asd = _tc_post1(u, Ws1, enc1_b, wsd1)
    u = gat_enc(asd, x_out)
    y, st = _tc_ffbn_a(u, x_out, Ws1, enc1_b, ff1_W, ff1_b, 0.01)
    x1, asd = _tc_bn_b(y, st, bn1_g, bn1_b, wsd2)
    u = gat_enc(asd, x1)
    y, st = _tc_ffbn_a(u, x1, Ws2, enc2_b, ff2_W, ff2_b, 0.0)
    x2, asd = _tc_bn_b(y, st, bn2_g, bn2_b, wsd3)
    u = gat_enc(asd, x2)
    y, st = _tc_ffbn_a(u, x2, Ws3, enc3_b, ff3_W, ff3_b, 0.0)
    x3, asd_dec, hpad = _tc_bn_b(y, st, bn3_g, bn3_b, wsd_dec, whp)
    u_dec = _gat_sc(1, 16)(_pad_rows(asd_dec), _pad_rows(hpad),
                        srcs_p, dsts_p, rs3, msinit, uinit_dec)
    out = u_dec[:N, :1] + dec_b
    batch3 = batch.astype(_i32).reshape(N // BR, 1, BR)
    sums, cnt, valo = _tc_pool_value(x3, batch3, val_W1, val_b1, W2p, val_b2)
    value = valo[:, :1]
    return out, value


# speedup vs baseline: 16.6544x; 2.0802x over previous
import functools

import jax
import jax.numpy as jnp
from jax import lax
from jax.experimental import pallas as pl
from jax.experimental.pallas import tpu as pltpu
from jax.experimental.pallas import tpu_sc as plsc

N = 10000
D = 128
NG = 16
NH = 8
SUBD = 64
NSUB = 160
NPAD = NSUB * SUBD
NW = 32
SPW = NSUB // NW
EB = 128
ETOT = 160000 + N
EPAD = ((ETOT + EB - 1) // EB) * EB
RSP = 168
BR = 400

_f32 = jnp.float32
_i32 = jnp.int32


def _lanesplat(vec, h):
    idx = jnp.full((16, 1), h, _i32)
    dn = lax.GatherDimensionNumbers(
        offset_dims=(), collapsed_slice_dims=(0,), start_index_map=(0,))
    return lax.gather(vec, idx, dn, (1,),
                      mode=lax.GatherScatterMode.PROMISE_IN_BOUNDS)


def _make_gat_sc(H, C):
    HC = H * C
    mesh = plsc.VectorSubcoreMesh(core_axis_name="c", subcore_axis_name="s",
                                  num_cores=2, num_subcores=16)

    def body(asd_hbm, xin_hbm, srcs_hbm, dsts_hbm, rs3_hbm,
             msinit_hbm, uinit_hbm, u_hbm,
             rs_v, srcs_v, dsts_v, asd_sg, adl_v, x_g, ms_v, u_v, sem):
        w = lax.axis_index("c") * 16 + lax.axis_index("s")
        pltpu.sync_copy(rs3_hbm, rs_v)
        rsw = rs_v[pl.ds(w * 16, 16)]

        for sci in range(SPW):
            sc = w * SPW + sci
            d0 = sc * SUBD
            e_lo = rsw[sci]
            e_hi = rsw[sci + 1]
            b_lo = e_lo // EB
            b_hi = (e_hi + EB - 1) // EB
            pltpu.sync_copy(msinit_hbm, ms_v)
            pltpu.sync_copy(uinit_hbm, u_v)
            pltpu.sync_copy(asd_hbm.at[pl.ds(d0, SUBD)], adl_v)

            def p_batch(b, carry):
                pltpu.sync_copy(srcs_hbm.at[pl.ds(b * EB, EB)], srcs_v)
                pltpu.sync_copy(dsts_hbm.at[pl.ds(b * EB, EB)],
                                dsts_v.at[pl.ds(0, EB)])
                pltpu.async_copy(asd_hbm.at[srcs_v], asd_sg, sem).wait()
                pltpu.async_copy(xin_hbm.at[srcs_v], x_g, sem).wait()

                def p_edge(j, c2):
                    dl = dsts_v[pl.ds(j, 16)][0] - d0

                    @pl.when((dl >= 0) & (dl < SUBD))
                    def _():
                        lv = asd_sg[j, 0:16] + adl_v[dl, 16:32]
                        lv = jnp.maximum(lv, 0.2 * lv)
                        mold = ms_v[dl, 0:16]
                        sold = ms_v[dl, 16:32]
                        mnew = jnp.maximum(mold, lv)
                        scale = jnp.exp(mold - mnew)
                        p = jnp.exp(lv - mnew)
                        ms_v[dl, 0:16] = mnew
                        ms_v[dl, 16:32] = sold * scale + p

                        sb = [_lanesplat(scale, h) for h in range(H)]
                        pb = [_lanesplat(p, h) for h in range(H)]
                        for c in range(C // 16):
                            co = c * 16
                            xv = x_g[j, co:co + 16]
                            for h in range(H):
                                hco = h * C + co
                                u_v[dl, hco:hco + 16] = (
                                    u_v[dl, hco:hco + 16] * sb[h]
                                    + pb[h] * xv)
                    return c2

                return lax.fori_loop(0, EB, p_edge, carry)

            lax.fori_loop(b_lo, b_hi, p_batch, 0)

            def norm_row(dd, carry):
                rvec = 1.0 / (ms_v[dd, 16:32] + 1e-16)
                rb = [_lanesplat(rvec, h) for h in range(H)]
                for h in range(H):
                    for c in range(C // 16):
                        hco = h * C + c * 16
                        u_v[dd, hco:hco + 16] *= rb[h]
                return carry

            lax.fori_loop(0, SUBD, norm_row, 0)
            pltpu.sync_copy(u_v, u_hbm.at[pl.ds(d0, SUBD)])

    return pl.kernel(
        body,
        out_type=jax.ShapeDtypeStruct((NPAD, HC), _f32),
        mesh=mesh,
        scratch_types=[
            pltpu.VMEM((NW * 16,), _i32),
            pltpu.VMEM((EB,), _i32),
            pltpu.VMEM((EB + 16,), _i32),
            pltpu.VMEM((EB, D), _f32),
            pltpu.VMEM((SUBD, D), _f32),
            pltpu.VMEM((EB, D), _f32),
            pltpu.VMEM((SUBD, 32), _f32),
            pltpu.VMEM((SUBD, HC), _f32),
            pltpu.SemaphoreType.DMA,
        ],
    )


@functools.lru_cache(maxsize=None)
def _gat_sc(H, C):
    return _make_gat_sc(H, C)


def _dot(a, b):
    return jax.lax.dot_general(a, b, (((1,), (0,)), ((), ())),
                               preferred_element_type=_f32)


def _tc_emb(x, W, b, wsd):
    def body(x_ref, W_ref, b_ref, wsd_ref, xe_ref, asd_ref):
        t = _dot(x_ref[...], W_ref[...]) + b_ref[...]
        xe = jnp.where(t >= 0, t, 0.01 * t)
        xe_ref[...] = xe
        asd_ref[...] = _dot(xe, wsd_ref[...])

    return pl.pallas_call(
        body, grid=(N // BR,),
        in_specs=[pl.BlockSpec((BR, D), lambda i: (i, 0)),
                  pl.BlockSpec((D, D), lambda i: (0, 0)),
                  pl.BlockSpec((D,), lambda i: (0,)),
                  pl.BlockSpec((D, D), lambda i: (0, 0))],
        out_specs=[pl.BlockSpec((BR, D), lambda i: (i, 0)),
                   pl.BlockSpec((BR, D), lambda i: (i, 0))],
        out_shape=[jax.ShapeDtypeStruct((N, D), _f32),
                   jax.ShapeDtypeStruct((N, D), _f32)],
    )(x, W, b, wsd)


def _tc_post1(u, Ws, b, wsd):
    def body(u_ref, Ws_ref, b_ref, wsd_ref, xo_ref, asd_ref):
        g = _dot(u_ref[...], Ws_ref[...]) * (1.0 / NH) + b_ref[...]
        xo_ref[...] = g
        asd_ref[...] = _dot(g, wsd_ref[...])

    return pl.pallas_call(
        body, grid=(N // BR,),
        in_specs=[pl.BlockSpec((BR, NH * D), lambda i: (i, 0)),
                  pl.BlockSpec((NH * D, D), lambda i: (0, 0)),
                  pl.BlockSpec((D,), lambda i: (0,)),
                  pl.BlockSpec((D, D), lambda i: (0, 0))],
        out_specs=[pl.BlockSpec((BR, D), lambda i: (i, 0)),
                   pl.BlockSpec((BR, D), lambda i: (i, 0))],
        out_shape=[jax.ShapeDtypeStruct((N, D), _f32),
                   jax.ShapeDtypeStruct((N, D), _f32)],
    )(u, Ws, b, wsd)


def _tc_ffbn_a(u, x_in, Ws, b, ffW, ffb, slope):
    def body(u_ref, xin_ref, Ws_ref, b_ref, ffW_ref, ffb_ref, y_ref, st_ref):
        g = _dot(u_ref[...], Ws_ref[...]) * (1.0 / NH) + b_ref[...]
        t = g + xin_ref[...]
        z = _dot(t, ffW_ref[...]) + ffb_ref[...]
        a = jnp.where(z >= 0, z, slope * z)
        y = a + xin_ref[...]
        y_ref[...] = y

        @pl.when(pl.program_id(0) == 0)
        def _():
            st_ref[...] = jnp.zeros_like(st_ref)

        st_ref[0:1, :] += jnp.sum(y, axis=0)[None, :]
        st_ref[1:2, :] += jnp.sum(y * y, axis=0)[None, :]

    return pl.pallas_call(
        body, grid=(N // BR,),
        in_specs=[pl.BlockSpec((BR, NH * D), lambda i: (i, 0)),
                  pl.BlockSpec((BR, D), lambda i: (i, 0)),
                  pl.BlockSpec((NH * D, D), lambda i: (0, 0)),
                  pl.BlockSpec((D,), lambda i: (0,)),
                  pl.BlockSpec((D, D), lambda i: (0, 0)),
                  pl.BlockSpec((D,), lambda i: (0,))],
        out_specs=[pl.BlockSpec((BR, D), lambda i: (i, 0)),
                   pl.BlockSpec((8, D), lambda i: (0, 0))],
        out_shape=[jax.ShapeDtypeStruct((N, D), _f32),
                   jax.ShapeDtypeStruct((8, D), _f32)],
    )(u, x_in, Ws, b, ffW, ffb)


def _tc_bn_b(y, st, g, b, wsd, whp=None):
    nout = 3 if whp is not None else 2

    def body(*refs):
        if nout == 3:
            (y_ref, st_ref, g_ref, b_ref, wsd_ref, whp_ref,
             xn_ref, asd_ref, hp_ref) = refs
        else:
            y_ref, st_ref, g_ref, b_ref, wsd_ref, xn_ref, asd_ref = refs
        m = st_ref[0:1, :] * (1.0 / N)
        v = st_ref[1:2, :] * (1.0 / N) - m * m
        xn = (y_ref[...] - m) * lax.rsqrt(v + 1e-5) * g_ref[...] + b_ref[...]
        xn_ref[...] = xn
        asd_ref[...] = _dot(xn, wsd_ref[...])
        if nout == 3:
            hp_ref[...] = _dot(xn, whp_ref[...])

    in_specs = [pl.BlockSpec((BR, D), lambda i: (i, 0)),
                pl.BlockSpec((8, D), lambda i: (0, 0)),
                pl.BlockSpec((D,), lambda i: (0,)),
                pl.BlockSpec((D,), lambda i: (0,)),
                pl.BlockSpec((D, D), lambda i: (0, 0))]
    out_specs = [pl.BlockSpec((BR, D), lambda i: (i, 0)),
                 pl.BlockSpec((BR, D), lambda i: (i, 0))]
    out_shape = [jax.ShapeDtypeStruct((N, D), _f32),
                 jax.ShapeDtypeStruct((N, D), _f32)]
    args = [y, st, g, b, wsd]
    if whp is not None:
        in_specs.append(pl.BlockSpec((D, D), lambda i: (0, 0)))
        out_specs.append(pl.BlockSpec((BR, D), lambda i: (i, 0)))
        out_shape.append(jax.ShapeDtypeStruct((N, D), _f32))
        args.append(whp)
    return pl.pallas_call(
        body, grid=(N // BR,), in_specs=in_specs,
        out_specs=out_specs, out_shape=out_shape)(*args)


def _tc_pool_value(x3, batch3, W1, b1, W2p, b2):
    nb = N // BR

    def body(x_ref, b_ref, W1_ref, b1_ref, W2_ref, b2_ref,
             sums_ref, cnt_ref, val_ref):
        @pl.when(pl.program_id(0) == 0)
        def _():
            sums_ref[...] = jnp.zeros_like(sums_ref)
            cnt_ref[...] = jnp.zeros_like(cnt_ref)

        bvec = b_ref[0, 0, :]
        S = (bvec[None, :] ==
             lax.broadcasted_iota(_i32, (NG, BR), 0)).astype(_f32)
        sums_ref[...] += _dot(S, x_ref[...])
        cnt_ref[...] += jnp.broadcast_to(jnp.sum(S, axis=1)[:, None], (NG, D))

        @pl.when(pl.program_id(0) == nb - 1)
        def _():
            vin = sums_ref[...] / jnp.maximum(cnt_ref[...], 1.0)
            hh = jnp.maximum(_dot(vin, W1_ref[...]) + b1_ref[...], 0.0)
            val_ref[...] = _dot(hh, W2_ref[...]) + b2_ref[...]

    return pl.pallas_call(
        body, grid=(nb,),
        in_specs=[pl.BlockSpec((BR, D), lambda i: (i, 0)),
                  pl.BlockSpec((1, 1, BR), lambda i: (i, 0, 0)),
                  pl.BlockSpec((D, D), lambda i: (0, 0)),
                  pl.BlockSpec((D,), lambda i: (0,)),
                  pl.BlockSpec((D, D), lambda i: (0, 0)),
                  pl.BlockSpec((1,), lambda i: (0,))],
        out_specs=[pl.BlockSpec((NG, D), lambda i: (0, 0)),
                   pl.BlockSpec((NG, D), lambda i: (0, 0)),
                   pl.BlockSpec((NG, D), lambda i: (0, 0))],
        out_shape=[jax.ShapeDtypeStruct((NG, D), _f32),
                   jax.ShapeDtypeStruct((NG, D), _f32),
                   jax.ShapeDtypeStruct((NG, D), _f32)],
    )(x3, batch3, W1, b1, W2p, b2)


def _enc_weights(W, a_src, a_dst):
    Wr = W.reshape(D, NH, D)
    ws = jnp.einsum('ikj,kj->ik', Wr, a_src)
    wd = jnp.einsum('ikj,kj->ik', Wr, a_dst)
    wsd = jnp.zeros((D, D), _f32)
    wsd = wsd.at[:, 0:NH].set(ws).at[:, 16:16 + NH].set(wd)
    Wstack = Wr.transpose(1, 0, 2).reshape(NH * D, D)
    return wsd, Wstack


def _pad_rows(a):
    return jnp.pad(a, ((0, NPAD - N), (0, 0)))


def kernel(x, emb_W, emb_b, enc1_W, enc1_asrc, enc1_adst, enc1_b, ff1_W,
           ff1_b, bn1_g, bn1_b, enc2_W, enc2_asrc, enc2_adst, enc2_b, ff2_W,
           ff2_b, bn2_g, bn2_b, enc3_W, enc3_asrc, enc3_adst, enc3_b, ff3_W,
           ff3_b, bn3_g, bn3_b, dec_W, dec_asrc, dec_adst, dec_b, val_W1,
           val_b1, val_W2, val_b2, edge_index, batch):
    loop = jnp.arange(N, dtype=edge_index.dtype)
    srcf = jnp.concatenate([edge_index[0], loop])
    dstf = jnp.concatenate([edge_index[1], loop])
    dsts, srcs = lax.sort([dstf, srcf], num_keys=1)
    srcs_p = jnp.pad(srcs, (0, EPAD - ETOT)).astype(_i32)
    dsts_p = jnp.pad(dsts, (0, EPAD - ETOT),
                     constant_values=NPAD - 1).astype(_i32)
    bounds = jnp.arange(NSUB + 1, dtype=_i32) * SUBD
    rs = jnp.searchsorted(dsts_p, bounds).astype(_i32)
    rs_idx = jnp.minimum(
        jnp.arange(NW, dtype=_i32)[:, None] * SPW
        + jnp.arange(16, dtype=_i32)[None, :], NSUB)
    rs3 = rs[rs_idx].reshape(-1)
    msinit = jnp.concatenate(
        [jnp.full((SUBD, 16), -1e30, _f32), jnp.zeros((SUBD, 16), _f32)],
        axis=1)
    uinit_enc = jnp.zeros((SUBD, NH * D), _f32)
    uinit_dec = jnp.zeros((SUBD, 16), _f32)

    wsd1, Ws1 = _enc_weights(enc1_W, enc1_asrc, enc1_adst)
    wsd2, Ws2 = _enc_weights(enc2_W, enc2_asrc, enc2_adst)
    wsd3, Ws3 = _enc_weights(enc3_W, enc3_asrc, enc3_adst)
    wsd_dec = jnp.zeros((D, D), _f32)
    wsd_dec = wsd_dec.at[:, 0].set(dec_W[:, 0] * dec_asrc[0, 0])
    wsd_dec = wsd_dec.at[:, 16].set(dec_W[:, 0] * dec_adst[0, 0])
    whp = jnp.zeros((D, D), _f32).at[:, 0].set(dec_W[:, 0])
    W2p = jnp.pad(val_W2, ((0, 0), (0, D - val_W2.shape[1])))

    def gat_enc(asd, xin):
        u = _gat_sc(NH, D)(_pad_rows(asd), _pad_rows(xin),
                        srcs_p, dsts_p, rs3, msinit, uinit_enc)
        return u[:N]

    x_e, asd = _tc_emb(x, emb_W, emb_b, wsd1)
    u = gat_enc(asd, x_e)
    x_out, asd = _tc_post1(u, Ws1, enc1_b, wsd1)
    u = gat_enc(asd, x_out)
    y, st = _tc_ffbn_a(u, x_out, Ws1, enc1_b, ff1_W, ff1_b, 0.01)
    x1, asd = _tc_bn_b(y, st, bn1_g, bn1_b, wsd2)
    u = gat_enc(asd, x1)
    y, st = _tc_ffbn_a(u, x1, Ws2, enc2_b, ff2_W, ff2_b, 0.0)
    x2, asd = _tc_bn_b(y, st, bn2_g, bn2_b, wsd3)
    u = gat_enc(asd, x2)
    y, st = _tc_ffbn_a(u, x2, Ws3, enc3_b, ff3_W, ff3_b, 0.0)
    x3, asd_dec, hpad = _tc_bn_b(y, st, bn3_g, bn3_b, wsd_dec, whp)
    u_dec = _gat_sc(1, 16)(_pad_rows(asd_dec), _pad_rows(hpad),
                        srcs_p, dsts_p, rs3, msinit, uinit_dec)
    out = u_dec[:N, :1] + dec_b
    batch3 = batch.astype(_i32).reshape(N // BR, 1, BR)
    sums, cnt, valo = _tc_pool_value(x3, batch3, val_W1, val_b1, W2p, val_b2)
    value = valo[:, :1]
    return out, value
